# Initial kernel scaffold; baseline (speedup 1.0000x reference)
import functools

import jax
import jax.numpy as jnp
from jax import lax
from jax.experimental import pallas as pl
from jax.experimental.pallas import tpu as pltpu
from jax.experimental.pallas import tpu_sc as plsc

NUM_CORES = 2
NUM_SUBCORES = 16
NUM_WORKERS = NUM_CORES * NUM_SUBCORES
LANES = 128
IDX_ROWS = 8


def _sc_mesh():
    return plsc.VectorSubcoreMesh(core_axis_name="c", subcore_axis_name="s")


def _make_deg_kernel(rows2d, np_pad):
    rows_per_worker = rows2d // NUM_WORKERS
    steps = rows_per_worker // IDX_ROWS
    tpt = np_pad // NUM_SUBCORES

    @functools.partial(
        pl.kernel,
        mesh=_sc_mesh(),
        out_type=jax.ShapeDtypeStruct((NUM_CORES, np_pad, 1), jnp.float32),
        scratch_types=[
            pltpu.VMEM_SHARED((np_pad, 1), jnp.float32),
            pltpu.VMEM((IDX_ROWS, LANES), jnp.int32),
            pltpu.VMEM((LANES, 1), jnp.float32),
        ],
    )
    def deg_kernel(dst_hbm, ones_hbm, zeros_hbm, deg_out, deg_sh, idx_v, ones_v):
        c = lax.axis_index("c")
        s = lax.axis_index("s")
        wid = c * NUM_SUBCORES + s
        sl = pl.ds(s * tpt, tpt)
        pltpu.sync_copy(zeros_hbm.at[sl, :], deg_sh.at[sl, :])
        pltpu.sync_copy(ones_hbm, ones_v)
        plsc.subcore_barrier()
        base = wid * rows_per_worker

        def step(o, carry):
            pltpu.sync_copy(dst_hbm.at[pl.ds(base + o * IDX_ROWS, IDX_ROWS), :], idx_v)
            for j in range(IDX_ROWS):
                pltpu.sync_copy(ones_v, deg_sh.at[idx_v.at[j]], add=True)
            return carry

        lax.fori_loop(0, steps, step, 0)
        plsc.subcore_barrier()
        pltpu.sync_copy(deg_sh.at[sl, :], deg_out.at[c, sl, :])

    return deg_kernel


def _make_prop_kernel(rows2d, np_pad, d):
    rows_per_worker = rows2d // NUM_WORKERS
    steps = rows_per_worker // IDX_ROWS
    tpt = np_pad // NUM_SUBCORES

    @functools.partial(
        pl.kernel,
        mesh=_sc_mesh(),
        out_type=jax.ShapeDtypeStruct((NUM_CORES, np_pad, d), jnp.float32),
        scratch_types=[
            pltpu.VMEM_SHARED((np_pad, d), jnp.float32),
            pltpu.VMEM_SHARED((np_pad, d), jnp.float32),
            pltpu.VMEM((IDX_ROWS, LANES), jnp.int32),
            pltpu.VMEM((IDX_ROWS, LANES), jnp.int32),
            pltpu.VMEM((LANES, d), jnp.float32),
        ],
    )
    def prop_kernel(src_hbm, dst_hbm, tab_hbm, zeros_hbm, acc_out,
                    tab_sh, acc_sh, idxs_v, idxd_v, rows_v):
        c = lax.axis_index("c")
        s = lax.axis_index("s")
        wid = c * NUM_SUBCORES + s
        sl = pl.ds(s * tpt, tpt)
        pltpu.sync_copy(tab_hbm.at[sl, :], tab_sh.at[sl, :])
        pltpu.sync_copy(zeros_hbm.at[sl, :], acc_sh.at[sl, :])
        plsc.subcore_barrier()
        base = wid * rows_per_worker

        def step(o, carry):
            row = pl.ds(base + o * IDX_ROWS, IDX_ROWS)
            pltpu.sync_copy(src_hbm.at[row, :], idxs_v)
            pltpu.sync_copy(dst_hbm.at[row, :], idxd_v)
            for j in range(IDX_ROWS):
                pltpu.sync_copy(tab_sh.at[idxs_v.at[j]], rows_v)
                pltpu.sync_copy(rows_v, acc_sh.at[idxd_v.at[j]], add=True)
            return carry

        lax.fori_loop(0, steps, step, 0)
        plsc.subcore_barrier()
        pltpu.sync_copy(acc_sh.at[sl, :], acc_out.at[c, sl, :])

    return prop_kernel


def _tc1_body(deg_ref, x4_ref, xs4_ref, dis_ref):
    d = deg_ref[0] + deg_ref[1] + 1.0
    dis = lax.rsqrt(d)
    xs4_ref[...] = x4_ref[...] * dis
    dis_ref[...] = dis


def _tc2_body(p_ref, xs4_ref, dis_ref, w1_ref, b1_ref, w2_ref, gs8_ref):
    dis = dis_ref[...]
    t4 = (p_ref[0] + p_ref[1] + xs4_ref[...]) * dis
    h = jnp.dot(t4, w1_ref[...], preferred_element_type=jnp.float32) + b1_ref[...]
    h = jnp.maximum(h, 0.0)
    g8 = jnp.dot(h, w2_ref[...], preferred_element_type=jnp.float32)
    gs8_ref[...] = g8 * dis


def _tc3_body(q_ref, gs8_ref, dis_ref, b2_ref, out_ref):
    u = (q_ref[0] + q_ref[1] + gs8_ref[...]) * dis_ref[...] + b2_ref[...]
    z = u[:, :7]
    m = jnp.max(z, axis=1, keepdims=True)
    e = jnp.exp(z - m)
    out_ref[...] = z - m - jnp.log(jnp.sum(e, axis=1, keepdims=True))


def kernel(x, edge_index, W1, b1, W2, b2):
    n = x.shape[0]
    e = edge_index.shape[1]
    np_pad = ((n + 96 + 127) // 128) * 128
    epw = NUM_WORKERS * LANES * IDX_ROWS
    e_pad = ((e + epw - 1) // epw) * epw
    rows2d = e_pad // LANES

    src = edge_index[0].astype(jnp.int32)
    dst = edge_index[1].astype(jnp.int32)
    sent = (jnp.arange(e_pad - e, dtype=jnp.int32) % 96) + n
    src2d = jnp.concatenate([src, sent]).reshape(rows2d, LANES)
    dst2d = jnp.concatenate([dst, sent]).reshape(rows2d, LANES)

    x4 = jnp.pad(x, ((0, np_pad - n), (0, 1)))
    w1p = jnp.pad(W1, ((0, 1), (0, 0)))
    w2p = jnp.pad(W2, ((0, 0), (0, 1)))
    b1r = b1.reshape(1, 16)
    b2p = jnp.pad(b2, (0, 1)).reshape(1, 8)
    ones128 = jnp.ones((LANES, 1), jnp.float32)
    z1 = jnp.zeros((np_pad, 1), jnp.float32)
    z4 = jnp.zeros((np_pad, 4), jnp.float32)
    z8 = jnp.zeros((np_pad, 8), jnp.float32)

    deg2 = _make_deg_kernel(rows2d, np_pad)(dst2d, ones128, z1)

    grid = 16
    r = np_pad // grid
    xs4, dis = pl.pallas_call(
        _tc1_body,
        grid=(grid,),
        in_specs=[
            pl.BlockSpec((2, r, 1), lambda i: (0, i, 0)),
            pl.BlockSpec((r, 4), lambda i: (i, 0)),
        ],
        out_specs=[
            pl.BlockSpec((r, 4), lambda i: (i, 0)),
            pl.BlockSpec((r, 1), lambda i: (i, 0)),
        ],
        out_shape=[
            jax.ShapeDtypeStruct((np_pad, 4), jnp.float32),
            jax.ShapeDtypeStruct((np_pad, 1), jnp.float32),
        ],
    )(deg2, x4)

    p2 = _make_prop_kernel(rows2d, np_pad, 4)(src2d, dst2d, xs4, z4)

    gs8 = pl.pallas_call(
        _tc2_body,
        grid=(grid,),
        in_specs=[
            pl.BlockSpec((2, r, 4), lambda i: (0, i, 0)),
            pl.BlockSpec((r, 4), lambda i: (i, 0)),
            pl.BlockSpec((r, 1), lambda i: (i, 0)),
            pl.BlockSpec((4, 16), lambda i: (0, 0)),
            pl.BlockSpec((1, 16), lambda i: (0, 0)),
            pl.BlockSpec((16, 8), lambda i: (0, 0)),
        ],
        out_specs=pl.BlockSpec((r, 8), lambda i: (i, 0)),
        out_shape=jax.ShapeDtypeStruct((np_pad, 8), jnp.float32),
    )(p2, xs4, dis, w1p, b1r, w2p)

    q2 = _make_prop_kernel(rows2d, np_pad, 8)(src2d, dst2d, gs8, z8)

    out = pl.pallas_call(
        _tc3_body,
        grid=(grid,),
        in_specs=[
            pl.BlockSpec((2, r, 8), lambda i: (0, i, 0)),
            pl.BlockSpec((r, 8), lambda i: (i, 0)),
            pl.BlockSpec((r, 1), lambda i: (i, 0)),
            pl.BlockSpec((1, 8), lambda i: (0, 0)),
        ],
        out_specs=pl.BlockSpec((r, 7), lambda i: (i, 0)),
        out_shape=jax.ShapeDtypeStruct((np_pad, 7), jnp.float32),
    )(q2, gs8, dis, b2p)

    return out[:n]


# speedup vs baseline: 42.4037x; 42.4037x over previous
import functools

import jax
import jax.numpy as jnp
from jax import lax
from jax.experimental import pallas as pl
from jax.experimental.pallas import tpu as pltpu
from jax.experimental.pallas import tpu_sc as plsc

NUM_CORES = 2
NUM_SUBCORES = 16
NUM_WORKERS = NUM_CORES * NUM_SUBCORES
LANES = 512

_SC_PARAMS = dict(
    mesh=plsc.VectorSubcoreMesh(core_axis_name="c", subcore_axis_name="s", num_cores=NUM_CORES, num_subcores=NUM_SUBCORES),
    compiler_params=pltpu.CompilerParams(needs_layout_passes=False),
)


def _make_deg_kernel(e_pad, np_pad):
    epw = e_pad // NUM_WORKERS
    windows = epw // LANES
    tpt = np_pad // NUM_SUBCORES

    @functools.partial(
        pl.kernel,
        out_type=jax.ShapeDtypeStruct((NUM_CORES, np_pad), jnp.float32),
        scratch_types=[
            pltpu.VMEM_SHARED((np_pad,), jnp.float32),
            pltpu.VMEM((LANES,), jnp.int32),
            pltpu.VMEM((LANES,), jnp.float32),
        ],
        **_SC_PARAMS,
    )
    def deg_kernel(dst_hbm, ones_hbm, zeros_hbm, deg_out, deg_sh, idx_v, ones_v):
        c = lax.axis_index("c")
        s = lax.axis_index("s")
        wid = c * NUM_SUBCORES + s
        sl = pl.ds(s * tpt, tpt)
        pltpu.sync_copy(zeros_hbm.at[sl], deg_sh.at[sl])
        pltpu.sync_copy(ones_hbm, ones_v)
        plsc.subcore_barrier()
        base = wid * epw

        def step(o, carry):
            pltpu.sync_copy(dst_hbm.at[pl.ds(base + o * LANES, LANES)], idx_v)
            pltpu.sync_copy(ones_v, deg_sh.at[idx_v], add=True)
            return carry

        lax.fori_loop(0, windows, step, 0)
        plsc.subcore_barrier()
        pltpu.sync_copy(deg_sh.at[sl], deg_out.at[c, sl])

    return deg_kernel


def _make_prop_kernel(e_pad, np_pad, d):
    epw = e_pad // NUM_WORKERS
    windows = epw // LANES
    tpt = np_pad // NUM_SUBCORES

    @functools.partial(
        pl.kernel,
        out_type=jax.ShapeDtypeStruct((NUM_CORES, d, np_pad), jnp.float32),
        scratch_types=(
            [pltpu.VMEM_SHARED((np_pad,), jnp.float32) for _ in range(2 * d)]
            + [pltpu.VMEM((LANES,), jnp.int32) for _ in range(2)]
            + [pltpu.VMEM((LANES,), jnp.float32) for _ in range(d)]
        ),
        **_SC_PARAMS,
    )
    def prop_kernel(src_hbm, dst_hbm, tab_hbm, zeros_hbm, acc_out, *scratch):
        tabs = scratch[:d]
        accs = scratch[d:2 * d]
        idxs_v, idxd_v = scratch[2 * d:2 * d + 2]
        cols = scratch[2 * d + 2:]
        c = lax.axis_index("c")
        s = lax.axis_index("s")
        wid = c * NUM_SUBCORES + s
        sl = pl.ds(s * tpt, tpt)
        for f in range(d):
            pltpu.sync_copy(tab_hbm.at[f, sl], tabs[f].at[sl])
            pltpu.sync_copy(zeros_hbm.at[sl], accs[f].at[sl])
        plsc.subcore_barrier()
        base = wid * epw

        def step(o, carry):
            win = pl.ds(base + o * LANES, LANES)
            pltpu.sync_copy(src_hbm.at[win], idxs_v)
            pltpu.sync_copy(dst_hbm.at[win], idxd_v)
            for f in range(d):
                pltpu.sync_copy(tabs[f].at[idxs_v], cols[f])
                pltpu.sync_copy(cols[f], accs[f].at[idxd_v], add=True)
            return carry

        lax.fori_loop(0, windows, step, 0)
        plsc.subcore_barrier()
        for f in range(d):
            pltpu.sync_copy(accs[f].at[sl], acc_out.at[c, f, sl])

    return prop_kernel


def _tc1_body(deg_ref, x4t_ref, xs4t_ref, dis_ref):
    dsum = deg_ref[0:1, :] + deg_ref[1:2, :] + 1.0
    dis = lax.rsqrt(dsum)
    xs4t_ref[...] = x4t_ref[...] * dis
    dis_ref[...] = dis


def _tc2_body(p_ref, xs4t_ref, dis_ref, w1t_ref, b1c_ref, w2t_ref, gs8t_ref):
    dis = dis_ref[...]
    t4 = (p_ref[0] + p_ref[1] + xs4t_ref[...]) * dis
    h = jnp.dot(w1t_ref[...], t4, preferred_element_type=jnp.float32) + b1c_ref[...]
    h = jnp.maximum(h, 0.0)
    g8 = jnp.dot(w2t_ref[...], h, preferred_element_type=jnp.float32)
    gs8t_ref[...] = g8 * dis


def _tc3_body(q_ref, gs8t_ref, dis_ref, b2c_ref, out_ref):
    u = (q_ref[0] + q_ref[1] + gs8t_ref[...]) * dis_ref[...]
    z = u[:7, :] + b2c_ref[...]
    m = jnp.max(z, axis=0, keepdims=True)
    e = jnp.exp(z - m)
    ls = z - m - jnp.log(jnp.sum(e, axis=0, keepdims=True))
    out_ref[...] = ls.T


def kernel(x, edge_index, W1, b1, W2, b2):
    n = x.shape[0]
    e = edge_index.shape[1]
    np_pad = ((n + 256 + 2047) // 2048) * 2048
    epw2 = NUM_WORKERS * LANES * 2
    e_pad = ((e + epw2 - 1) // epw2) * epw2

    src = edge_index[0].astype(jnp.int32)
    dst = edge_index[1].astype(jnp.int32)
    sent = (jnp.arange(e_pad - e, dtype=jnp.int32) % 256) + n
    src1d = jnp.concatenate([src, sent])
    dst1d = jnp.concatenate([dst, sent])

    x4t = jnp.pad(x, ((0, np_pad - n), (0, 1))).T
    w1t = jnp.pad(W1, ((0, 1), (0, 0))).T
    b1c = b1.reshape(16, 1)
    w2t = jnp.pad(W2, ((0, 0), (0, 1))).T
    b2c = b2.reshape(7, 1)
    ones128 = jnp.ones((LANES,), jnp.float32)
    z1 = jnp.zeros((np_pad,), jnp.float32)

    deg2 = _make_deg_kernel(e_pad, np_pad)(dst1d, ones128, z1)

    grid = 16
    r = np_pad // grid
    xs4t, dis = pl.pallas_call(
        _tc1_body,
        grid=(grid,),
        in_specs=[
            pl.BlockSpec((2, r), lambda i: (0, i)),
            pl.BlockSpec((4, r), lambda i: (0, i)),
        ],
        out_specs=[
            pl.BlockSpec((4, r), lambda i: (0, i)),
            pl.BlockSpec((1, r), lambda i: (0, i)),
        ],
        out_shape=[
            jax.ShapeDtypeStruct((4, np_pad), jnp.float32),
            jax.ShapeDtypeStruct((1, np_pad), jnp.float32),
        ],
    )(deg2, x4t)

    p2 = _make_prop_kernel(e_pad, np_pad, 4)(src1d, dst1d, xs4t, z1)

    gs8t = pl.pallas_call(
        _tc2_body,
        grid=(grid,),
        in_specs=[
            pl.BlockSpec((2, 4, r), lambda i: (0, 0, i)),
            pl.BlockSpec((4, r), lambda i: (0, i)),
            pl.BlockSpec((1, r), lambda i: (0, i)),
            pl.BlockSpec((16, 4), lambda i: (0, 0)),
            pl.BlockSpec((16, 1), lambda i: (0, 0)),
            pl.BlockSpec((8, 16), lambda i: (0, 0)),
        ],
        out_specs=pl.BlockSpec((8, r), lambda i: (0, i)),
        out_shape=jax.ShapeDtypeStruct((8, np_pad), jnp.float32),
    )(p2, xs4t, dis, w1t, b1c, w2t)

    q2 = _make_prop_kernel(e_pad, np_pad, 8)(src1d, dst1d, gs8t, z1)

    out = pl.pallas_call(
        _tc3_body,
        grid=(grid,),
        in_specs=[
            pl.BlockSpec((2, 8, r), lambda i: (0, 0, i)),
            pl.BlockSpec((8, r), lambda i: (0, i)),
            pl.BlockSpec((1, r), lambda i: (0, i)),
            pl.BlockSpec((7, 1), lambda i: (0, 0)),
        ],
        out_specs=pl.BlockSpec((r, 7), lambda i: (i, 0)),
        out_shape=jax.ShapeDtypeStruct((np_pad, 7), jnp.float32),
    )(q2, gs8t, dis, b2c)

    return out[:n]
